# baseline (speedup 1.0000x reference)
import functools

import jax
import jax.numpy as jnp
import numpy as np
from jax import lax
from jax.experimental import pallas as pl
from jax.experimental.pallas import tpu as pltpu
from jax.experimental.pallas import tpu_sc as plsc

_NC, _NS, _LANES = 2, 16, 16
_NW = _NC * _NS
_FB = 16384
_PIECE = 32768
_NBUF = 3


def _intervals_for(L, ratio=0.15, seed=0):
    rng = np.random.default_rng(seed)
    min_win, max_win = 0, int(0.05 * L)
    intervals, durations = [], []
    while sum(durations) < ratio * L:
        random_start = int(rng.integers(0, L - max_win))
        random_end = random_start + int(rng.integers(min_win, max_win))
        random_win = np.arange(random_start, random_end)
        intersections = [len(np.intersect1d(p, random_win)) for p in intervals]
        if sum(intersections) >= random_end - random_start:
            continue
        intervals.append(random_win)
        durations.append(random_end - random_start - sum(intersections))
    return intervals


@functools.lru_cache(maxsize=None)
def _runs_for(L):
    idx = np.arange(L, dtype=np.int64)
    for win in _intervals_for(L):
        src = idx[win[0] - 1] if win[0] else idx[0]
        idx[win] = src
    masked = np.flatnonzero(idx != np.arange(L))
    runs = []
    if masked.size:
        start = prev = int(masked[0])
        val = int(idx[start])
        for i in masked[1:]:
            i = int(i)
            if i == prev + 1 and int(idx[i]) == val:
                prev = i
            else:
                runs.append((start, prev + 1, val))
                start = prev = i
                val = int(idx[i])
        runs.append((start, prev + 1, val))
    return tuple(runs)


@functools.lru_cache(maxsize=None)
def _plans_for(L):
    runs = _runs_for(L)
    chunk = L // _NW
    assert L % _NW == 0 and chunk % _LANES == 0
    excluded = []
    rmw_all = []
    fill_all = []
    for r, (s, e, _) in enumerate(runs):
        f0 = s - s % _LANES
        f1 = e + (-e) % _LANES
        excluded.append((f0, f1))
        rmw_all.append((f0, s - f0, min(e - f0, _LANES), r))
        mid_lo = f0 + _LANES
        mid_hi = f1 - _LANES if e % _LANES else f1
        if e % _LANES and f1 - _LANES > f0:
            rmw_all.append((f1 - _LANES, 0, e - (f1 - _LANES), r))
        if mid_hi > mid_lo:
            fill_all.append((mid_lo, mid_hi, r))
    excluded.sort()
    for (a0, a1), (b0, b1) in zip(excluded, excluded[1:]):
        assert a1 < b0, "runs assumed non-adjacent at 16-lane granularity"
    plans = {}
    for w in range(_NW):
        c0, c1 = w * chunk, (w + 1) * chunk
        spans, pos = [], c0
        for f0, f1 in excluded:
            lo, hi = max(f0, c0), min(f1, c1)
            if lo < hi:
                if pos < lo:
                    spans.append((pos, lo - pos))
                pos = hi
        if pos < c1:
            spans.append((pos, c1 - pos))
        rmw = [t for t in rmw_all if c0 <= t[0] < c1]
        fills = []
        for lo, hi, r in fill_all:
            a, b = max(lo, c0), min(hi, c1)
            if a < b:
                fills.append((a, b, r))
        rset = sorted({t[3] for t in rmw} | {t[2] for t in fills})
        assert len(rset) <= 2
        plans[w] = dict(spans=spans, rmw=rmw, fills=fills, rset=rset)
    return plans


def _pieces_of(plan):
    pieces = []
    for a, n in plan["spans"]:
        off = a
        while off < a + n:
            m = min(_PIECE, a + n - off)
            pieces.append((off, m))
            off += m
    return pieces


def _sc_body(L, runs, plans, x_hbm, o_hbm, fb, bufs, srcv, resv, sems_in,
             sems_out, sem):
    wid = lax.axis_index("s") * _NC + lax.axis_index("c")
    lane = lax.iota(jnp.int32, _LANES)
    primed = {w: {} for w in range(_NW)}
    for w, plan in plans.items():
        pieces = _pieces_of(plan)
        if not pieces:
            continue

        @pl.when(wid == w)
        def _prime(w=w, pieces=pieces):
            for i, (off, n) in enumerate(pieces[:_NBUF]):
                primed[w][i] = pltpu.async_copy(
                    x_hbm.at[pl.ds(off, n)],
                    bufs[i % _NBUF].at[pl.ds(0, n)],
                    sems_in[i % _NBUF],
                )

    addr = jnp.int32(0)
    lidx = jnp.int32(0)
    for w, plan in plans.items():
        if plan["rset"]:
            (r,) = plan["rset"]
            src = runs[r][2]
            addr = jnp.where(wid == w, jnp.int32(src - src % _LANES), addr)
            lidx = jnp.where(wid == w, jnp.int32(src % _LANES), lidx)
    addr = pl.multiple_of(addr, _LANES)
    pltpu.sync_copy(x_hbm.at[pl.ds(addr, _LANES)], srcv)
    srcvec = srcv[...]
    idxvec = jnp.broadcast_to(lidx, (_LANES,)).astype(jnp.int32)
    dnums = lax.GatherDimensionNumbers(
        offset_dims=(), collapsed_slice_dims=(0,), start_index_map=(0,)
    )
    fv = lax.gather(
        srcvec,
        idxvec[:, None],
        dnums,
        (1,),
        mode=lax.GatherScatterMode.PROMISE_IN_BOUNDS,
    )
    for i in range(_FB // _LANES):
        fb[pl.ds(i * _LANES, _LANES)] = fv

    for w, plan in plans.items():
        pieces = _pieces_of(plan)

        @pl.when(wid == w)
        def _work(w=w, plan=plan, pieces=pieces):
            cps_in = dict(primed[w])
            cps_out = {}
            for i, (off, n) in enumerate(pieces):
                b = i % _NBUF
                if i >= _NBUF:
                    cps_out[i - _NBUF].wait()
                    cps_in[i] = pltpu.async_copy(
                        x_hbm.at[pl.ds(off, n)],
                        bufs[b].at[pl.ds(0, n)],
                        sems_in[b],
                    )
                cps_in[i].wait()
                cps_out[i] = pltpu.async_copy(
                    bufs[b].at[pl.ds(0, n)],
                    o_hbm.at[pl.ds(off, n)],
                    sems_out[b],
                )
            fill_copies = []
            for a, bb, _ in plan["fills"]:
                off = a
                while off < bb:
                    n = min(_FB, bb - off)
                    fill_copies.append(
                        pltpu.async_copy(
                            fb.at[pl.ds(0, n)], o_hbm.at[pl.ds(off, n)], sem
                        )
                    )
                    off += n
            for g, l0, l1, _ in plan["rmw"]:
                pltpu.sync_copy(x_hbm.at[pl.ds(g, _LANES)], resv)
                resv[...] = jnp.where((lane >= l0) & (lane < l1), fv, resv[...])
                pltpu.sync_copy(resv, o_hbm.at[pl.ds(g, _LANES)])
            for i in range(max(0, len(pieces) - _NBUF), len(pieces)):
                cps_out[i].wait()
            for cp in fill_copies:
                cp.wait()


def kernel(x):
    L = x.shape[-1]
    runs = _runs_for(L)
    plans = _plans_for(L)
    mesh = plsc.VectorSubcoreMesh(core_axis_name="c", subcore_axis_name="s")
    body = functools.partial(_sc_body, L, runs, plans)
    return pl.kernel(
        body,
        out_type=jax.ShapeDtypeStruct((L,), x.dtype),
        mesh=mesh,
        scratch_types=[
            pltpu.VMEM((_FB,), jnp.float32),
            [pltpu.VMEM((_PIECE,), jnp.float32) for _ in range(_NBUF)],
            pltpu.VMEM((_LANES,), jnp.float32),
            pltpu.VMEM((_LANES,), jnp.float32),
            [pltpu.SemaphoreType.DMA for _ in range(_NBUF)],
            [pltpu.SemaphoreType.DMA for _ in range(_NBUF)],
            pltpu.SemaphoreType.DMA,
        ],
    )(x)


# speedup vs baseline: 1.0261x; 1.0261x over previous
import functools

import jax
import jax.numpy as jnp
import numpy as np
from jax import lax
from jax.experimental import pallas as pl
from jax.experimental.pallas import tpu as pltpu
from jax.experimental.pallas import tpu_sc as plsc

_NC, _NS, _LANES = 2, 16, 16
_NW = _NC * _NS
_FB = 16384
_PIECE = 32768
_NBUF = 3


def _intervals_for(L, ratio=0.15, seed=0):
    rng = np.random.default_rng(seed)
    min_win, max_win = 0, int(0.05 * L)
    intervals, durations = [], []
    while sum(durations) < ratio * L:
        random_start = int(rng.integers(0, L - max_win))
        random_end = random_start + int(rng.integers(min_win, max_win))
        random_win = np.arange(random_start, random_end)
        intersections = [len(np.intersect1d(p, random_win)) for p in intervals]
        if sum(intersections) >= random_end - random_start:
            continue
        intervals.append(random_win)
        durations.append(random_end - random_start - sum(intersections))
    return intervals


@functools.lru_cache(maxsize=None)
def _runs_for(L):
    idx = np.arange(L, dtype=np.int64)
    for win in _intervals_for(L):
        src = idx[win[0] - 1] if win[0] else idx[0]
        idx[win] = src
    masked = np.flatnonzero(idx != np.arange(L))
    runs = []
    if masked.size:
        start = prev = int(masked[0])
        val = int(idx[start])
        for i in masked[1:]:
            i = int(i)
            if i == prev + 1 and int(idx[i]) == val:
                prev = i
            else:
                runs.append((start, prev + 1, val))
                start = prev = i
                val = int(idx[i])
        runs.append((start, prev + 1, val))
    return tuple(runs)


@functools.lru_cache(maxsize=None)
def _plans_for(L):
    runs = _runs_for(L)
    chunk = L // _NW
    assert L % _NW == 0 and chunk % _LANES == 0
    excluded = []
    rmw_all = []
    fill_all = []
    for r, (s, e, _) in enumerate(runs):
        f0 = s - s % _LANES
        f1 = e + (-e) % _LANES
        excluded.append((f0, f1))
        rmw_all.append((f0, s - f0, min(e - f0, _LANES), r))
        mid_lo = f0 + _LANES
        mid_hi = f1 - _LANES if e % _LANES else f1
        if e % _LANES and f1 - _LANES > f0:
            rmw_all.append((f1 - _LANES, 0, e - (f1 - _LANES), r))
        if mid_hi > mid_lo:
            fill_all.append((mid_lo, mid_hi, r))
    excluded.sort()
    for (a0, a1), (b0, b1) in zip(excluded, excluded[1:]):
        assert a1 < b0, "runs assumed non-adjacent at 16-lane granularity"
    plans = {}
    for w in range(_NW):
        c0, c1 = w * chunk, (w + 1) * chunk
        spans, pos = [], c0
        for f0, f1 in excluded:
            lo, hi = max(f0, c0), min(f1, c1)
            if lo < hi:
                if pos < lo:
                    spans.append((pos, lo - pos))
                pos = hi
        if pos < c1:
            spans.append((pos, c1 - pos))
        rmw = [t for t in rmw_all if c0 <= t[0] < c1]
        fills = []
        for lo, hi, r in fill_all:
            a, b = max(lo, c0), min(hi, c1)
            if a < b:
                fills.append((a, b, r))
        rset = sorted({t[3] for t in rmw} | {t[2] for t in fills})
        assert len(rset) <= 2
        plans[w] = dict(spans=spans, rmw=rmw, fills=fills, rset=rset)
    return plans


def _pieces_of(plan):
    pieces = []
    for a, n in plan["spans"]:
        off = a
        while off < a + n:
            m = min(_PIECE, a + n - off)
            pieces.append((off, m))
            off += m
    return pieces


def _sc_body(L, runs, plans, x_hbm, o_hbm, fb, bufs, srcv, resvs, sems_in,
             sems_out, sem, sems_rmw):
    wid = lax.axis_index("s") * _NC + lax.axis_index("c")
    lane = lax.iota(jnp.int32, _LANES)
    primed = {w: {} for w in range(_NW)}
    rmw_loads = {}
    for w, plan in plans.items():
        pieces = _pieces_of(plan)
        if not pieces and not plan["rmw"]:
            continue

        @pl.when(wid == w)
        def _prime(w=w, plan=plan, pieces=pieces):
            for i, (off, n) in enumerate(pieces[: _NBUF - 1]):
                primed[w][i] = pltpu.async_copy(
                    x_hbm.at[pl.ds(off, n)],
                    bufs[i % _NBUF].at[pl.ds(0, n)],
                    sems_in[i % _NBUF],
                )
            for k, (g, _, _, _) in enumerate(plan["rmw"]):
                rmw_loads[(w, k)] = pltpu.async_copy(
                    x_hbm.at[pl.ds(g, _LANES)], resvs[k], sems_rmw[k]
                )

    addr = jnp.int32(0)
    lidx = jnp.int32(0)
    for w, plan in plans.items():
        if plan["rset"]:
            (r,) = plan["rset"]
            src = runs[r][2]
            addr = jnp.where(wid == w, jnp.int32(src - src % _LANES), addr)
            lidx = jnp.where(wid == w, jnp.int32(src % _LANES), lidx)
    addr = pl.multiple_of(addr, _LANES)
    pltpu.sync_copy(x_hbm.at[pl.ds(addr, _LANES)], srcv)
    srcvec = srcv[...]
    idxvec = jnp.broadcast_to(lidx, (_LANES,)).astype(jnp.int32)
    dnums = lax.GatherDimensionNumbers(
        offset_dims=(), collapsed_slice_dims=(0,), start_index_map=(0,)
    )
    fv = lax.gather(
        srcvec,
        idxvec[:, None],
        dnums,
        (1,),
        mode=lax.GatherScatterMode.PROMISE_IN_BOUNDS,
    )
    for i in range(_FB // _LANES):
        fb[pl.ds(i * _LANES, _LANES)] = fv

    for w, plan in plans.items():
        pieces = _pieces_of(plan)

        @pl.when(wid == w)
        def _work(w=w, plan=plan, pieces=pieces):
            fill_copies = []
            for a, bb, _ in plan["fills"]:
                off = a
                while off < bb:
                    n = min(_FB, bb - off)
                    fill_copies.append(
                        pltpu.async_copy(
                            fb.at[pl.ds(0, n)], o_hbm.at[pl.ds(off, n)], sem
                        )
                    )
                    off += n
            cps_in = dict(primed[w])
            cps_out = {}
            waited = set()
            np_ = len(pieces)
            for i in range(np_):
                j = i + _NBUF - 1
                if j < np_:
                    if j >= _NBUF:
                        cps_out[j - _NBUF].wait()
                        waited.add(j - _NBUF)
                    off, n = pieces[j]
                    cps_in[j] = pltpu.async_copy(
                        x_hbm.at[pl.ds(off, n)],
                        bufs[j % _NBUF].at[pl.ds(0, n)],
                        sems_in[j % _NBUF],
                    )
                off, n = pieces[i]
                cps_in[i].wait()
                cps_out[i] = pltpu.async_copy(
                    bufs[i % _NBUF].at[pl.ds(0, n)],
                    o_hbm.at[pl.ds(off, n)],
                    sems_out[i % _NBUF],
                )
            for k, (g, l0, l1, _) in enumerate(plan["rmw"]):
                rmw_loads[(w, k)].wait()
                resvs[k][...] = jnp.where(
                    (lane >= l0) & (lane < l1), fv, resvs[k][...]
                )
                pltpu.sync_copy(resvs[k], o_hbm.at[pl.ds(g, _LANES)])
            for i in range(np_):
                if i not in waited:
                    cps_out[i].wait()
            for cp in fill_copies:
                cp.wait()


def kernel(x):
    L = x.shape[-1]
    runs = _runs_for(L)
    plans = _plans_for(L)
    mesh = plsc.VectorSubcoreMesh(core_axis_name="c", subcore_axis_name="s")
    body = functools.partial(_sc_body, L, runs, plans)
    return pl.kernel(
        body,
        out_type=jax.ShapeDtypeStruct((L,), x.dtype),
        mesh=mesh,
        scratch_types=[
            pltpu.VMEM((_FB,), jnp.float32),
            [pltpu.VMEM((_PIECE,), jnp.float32) for _ in range(_NBUF)],
            pltpu.VMEM((_LANES,), jnp.float32),
            [pltpu.VMEM((_LANES,), jnp.float32) for _ in range(2)],
            [pltpu.SemaphoreType.DMA for _ in range(_NBUF)],
            [pltpu.SemaphoreType.DMA for _ in range(_NBUF)],
            pltpu.SemaphoreType.DMA,
            [pltpu.SemaphoreType.DMA for _ in range(2)],
        ],
    )(x)


# speedup vs baseline: 2.2108x; 2.1545x over previous
import functools

import jax
import jax.numpy as jnp
import numpy as np
from jax.experimental import pallas as pl
from jax.experimental.pallas import tpu as pltpu

_BLOCK = 1024 * 1024


def _intervals_for(L, ratio=0.15, seed=0):
    rng = np.random.default_rng(seed)
    min_win, max_win = 0, int(0.05 * L)
    intervals, durations = [], []
    while sum(durations) < ratio * L:
        random_start = int(rng.integers(0, L - max_win))
        random_end = random_start + int(rng.integers(min_win, max_win))
        random_win = np.arange(random_start, random_end)
        intersections = [len(np.intersect1d(p, random_win)) for p in intervals]
        if sum(intersections) >= random_end - random_start:
            continue
        intervals.append(random_win)
        durations.append(random_end - random_start - sum(intersections))
    return intervals


@functools.lru_cache(maxsize=None)
def _runs_for(L):
    idx = np.arange(L, dtype=np.int64)
    for win in _intervals_for(L):
        src = idx[win[0] - 1] if win[0] else idx[0]
        idx[win] = src
    masked = np.flatnonzero(idx != np.arange(L))
    runs = []
    if masked.size:
        start = prev = int(masked[0])
        val = int(idx[start])
        for i in masked[1:]:
            i = int(i)
            if i == prev + 1 and int(idx[i]) == val:
                prev = i
            else:
                runs.append((start, prev + 1, val))
                start = prev = i
                val = int(idx[i])
        runs.append((start, prev + 1, val))
    return tuple(runs)


def _mask_body(block_fills, fills_ref, x_ref, o_ref):
    pid = pl.program_id(0)
    o_ref[...] = x_ref[...]
    for b, fills in block_fills.items():
        @pl.when(pid == b)
        def _fill(fills=fills):
            for ls, le, r in fills:
                o_ref[ls:le] = jnp.broadcast_to(fills_ref[r], (le - ls,))


def kernel(x):
    L = x.shape[-1]
    runs = _runs_for(L)
    grid = pl.cdiv(L, _BLOCK)
    block_fills = {}
    for r, (s, e, _) in enumerate(runs):
        for b in range(s // _BLOCK, (e - 1) // _BLOCK + 1):
            lo, hi = max(s, b * _BLOCK), min(e, (b + 1) * _BLOCK)
            block_fills.setdefault(b, []).append((lo - b * _BLOCK, hi - b * _BLOCK, r))
    srcs = jnp.asarray([src for (_, _, src) in runs], dtype=jnp.int32)
    nf = max(len(runs), 1)
    fills = x[srcs] if len(runs) else jnp.zeros((1,), x.dtype)
    out = pl.pallas_call(
        functools.partial(_mask_body, block_fills),
        grid=(grid,),
        in_specs=[
            pl.BlockSpec((nf,), lambda i: (0,)),
            pl.BlockSpec((_BLOCK,), lambda i: (i,)),
        ],
        out_specs=pl.BlockSpec((_BLOCK,), lambda i: (i,)),
        out_shape=jax.ShapeDtypeStruct((L,), x.dtype),
        compiler_params=pltpu.CompilerParams(
            dimension_semantics=("parallel",),
        ),
    )(fills, x)
    return out


# speedup vs baseline: 2.3221x; 1.0504x over previous
import functools

import jax
import jax.numpy as jnp
import numpy as np
from jax.experimental import pallas as pl
from jax.experimental.pallas import tpu as pltpu

_BLOCK = 2048 * 1024


def _intervals_for(L, ratio=0.15, seed=0):
    rng = np.random.default_rng(seed)
    min_win, max_win = 0, int(0.05 * L)
    intervals, durations = [], []
    while sum(durations) < ratio * L:
        random_start = int(rng.integers(0, L - max_win))
        random_end = random_start + int(rng.integers(min_win, max_win))
        random_win = np.arange(random_start, random_end)
        intersections = [len(np.intersect1d(p, random_win)) for p in intervals]
        if sum(intersections) >= random_end - random_start:
            continue
        intervals.append(random_win)
        durations.append(random_end - random_start - sum(intersections))
    return intervals


@functools.lru_cache(maxsize=None)
def _runs_for(L):
    idx = np.arange(L, dtype=np.int64)
    for win in _intervals_for(L):
        src = idx[win[0] - 1] if win[0] else idx[0]
        idx[win] = src
    masked = np.flatnonzero(idx != np.arange(L))
    runs = []
    if masked.size:
        start = prev = int(masked[0])
        val = int(idx[start])
        for i in masked[1:]:
            i = int(i)
            if i == prev + 1 and int(idx[i]) == val:
                prev = i
            else:
                runs.append((start, prev + 1, val))
                start = prev = i
                val = int(idx[i])
        runs.append((start, prev + 1, val))
    return tuple(runs)


def _mask_body(block_fills, fills_ref, x_ref, o_ref):
    pid = pl.program_id(0)
    o_ref[...] = x_ref[...]
    for b, fills in block_fills.items():
        @pl.when(pid == b)
        def _fill(fills=fills):
            for ls, le, r in fills:
                o_ref[ls:le] = jnp.broadcast_to(fills_ref[r], (le - ls,))


def kernel(x):
    L = x.shape[-1]
    runs = _runs_for(L)
    grid = pl.cdiv(L, _BLOCK)
    block_fills = {}
    for r, (s, e, _) in enumerate(runs):
        for b in range(s // _BLOCK, (e - 1) // _BLOCK + 1):
            lo, hi = max(s, b * _BLOCK), min(e, (b + 1) * _BLOCK)
            block_fills.setdefault(b, []).append((lo - b * _BLOCK, hi - b * _BLOCK, r))
    srcs = jnp.asarray([src for (_, _, src) in runs], dtype=jnp.int32)
    nf = max(len(runs), 1)
    fills = x[srcs] if len(runs) else jnp.zeros((1,), x.dtype)
    out = pl.pallas_call(
        functools.partial(_mask_body, block_fills),
        grid=(grid,),
        in_specs=[
            pl.BlockSpec((nf,), lambda i: (0,)),
            pl.BlockSpec((_BLOCK,), lambda i: (i,)),
        ],
        out_specs=pl.BlockSpec((_BLOCK,), lambda i: (i,)),
        out_shape=jax.ShapeDtypeStruct((L,), x.dtype),
        compiler_params=pltpu.CompilerParams(
            dimension_semantics=("parallel",),
        ),
    )(fills, x)
    return out


# speedup vs baseline: 2.3766x; 1.0234x over previous
import functools

import jax
import jax.numpy as jnp
import numpy as np
from jax.experimental import pallas as pl
from jax.experimental.pallas import tpu as pltpu

_BLOCK = 2731 * 1024


def _intervals_for(L, ratio=0.15, seed=0):
    rng = np.random.default_rng(seed)
    min_win, max_win = 0, int(0.05 * L)
    intervals, durations = [], []
    while sum(durations) < ratio * L:
        random_start = int(rng.integers(0, L - max_win))
        random_end = random_start + int(rng.integers(min_win, max_win))
        random_win = np.arange(random_start, random_end)
        intersections = [len(np.intersect1d(p, random_win)) for p in intervals]
        if sum(intersections) >= random_end - random_start:
            continue
        intervals.append(random_win)
        durations.append(random_end - random_start - sum(intersections))
    return intervals


@functools.lru_cache(maxsize=None)
def _runs_for(L):
    idx = np.arange(L, dtype=np.int64)
    for win in _intervals_for(L):
        src = idx[win[0] - 1] if win[0] else idx[0]
        idx[win] = src
    masked = np.flatnonzero(idx != np.arange(L))
    runs = []
    if masked.size:
        start = prev = int(masked[0])
        val = int(idx[start])
        for i in masked[1:]:
            i = int(i)
            if i == prev + 1 and int(idx[i]) == val:
                prev = i
            else:
                runs.append((start, prev + 1, val))
                start = prev = i
                val = int(idx[i])
        runs.append((start, prev + 1, val))
    return tuple(runs)


def _mask_body(block_fills, fills_ref, x_ref, o_ref):
    pid = pl.program_id(0)
    o_ref[...] = x_ref[...]
    for b, fills in block_fills.items():
        @pl.when(pid == b)
        def _fill(fills=fills):
            for ls, le, r in fills:
                o_ref[ls:le] = jnp.broadcast_to(fills_ref[r], (le - ls,))


def kernel(x):
    L = x.shape[-1]
    runs = _runs_for(L)
    grid = pl.cdiv(L, _BLOCK)
    block_fills = {}
    for r, (s, e, _) in enumerate(runs):
        for b in range(s // _BLOCK, (e - 1) // _BLOCK + 1):
            lo, hi = max(s, b * _BLOCK), min(e, (b + 1) * _BLOCK)
            block_fills.setdefault(b, []).append((lo - b * _BLOCK, hi - b * _BLOCK, r))
    srcs = jnp.asarray([src for (_, _, src) in runs], dtype=jnp.int32)
    nf = max(len(runs), 1)
    fills = x[srcs] if len(runs) else jnp.zeros((1,), x.dtype)
    out = pl.pallas_call(
        functools.partial(_mask_body, block_fills),
        grid=(grid,),
        in_specs=[
            pl.BlockSpec((nf,), lambda i: (0,)),
            pl.BlockSpec((_BLOCK,), lambda i: (i,)),
        ],
        out_specs=pl.BlockSpec((_BLOCK,), lambda i: (i,)),
        out_shape=jax.ShapeDtypeStruct((L,), x.dtype),
        compiler_params=pltpu.CompilerParams(
            dimension_semantics=("parallel",),
        ),
    )(fills, x)
    return out
